# baseline (device time: 11962 ns/iter reference)
import jax
import jax.numpy as jnp
from jax import lax
from jax.experimental import pallas as pl
from jax.experimental.pallas import tpu as pltpu

N_OUT = 512
NCHUNK = 4


def kernel(x):
    _, m, n_tot = x.shape
    rows = m // NCHUNK
    xb = x[0].astype(jnp.bfloat16)

    def body(x_ref, out_ref, recv_buf, send_sems, recv_sems):
        px = lax.axis_index("x")
        py = lax.axis_index("y")
        pz = lax.axis_index("z")
        partner = (1 - px, py, pz)

        barrier = pltpu.get_barrier_semaphore()
        pl.semaphore_signal(
            barrier, inc=1, device_id=partner,
            device_id_type=pl.DeviceIdType.MESH,
        )
        pl.semaphore_wait(barrier, 1)

        def rdma(i):
            sl = pl.ds(i * rows, rows)
            return pltpu.make_async_remote_copy(
                src_ref=x_ref.at[sl, pl.ds((1 - px) * N_OUT, N_OUT)],
                dst_ref=recv_buf.at[sl],
                send_sem=send_sems.at[i],
                recv_sem=recv_sems.at[i],
                device_id=partner,
                device_id_type=pl.DeviceIdType.MESH,
            )

        for i in range(NCHUNK):
            rdma(i).start()

        for i in range(NCHUNK):
            rdma(i).wait_recv()
            sl = pl.ds(i * rows, rows)
            out_ref[sl] = (
                x_ref[sl, pl.ds(px * N_OUT, N_OUT)].astype(jnp.float32)
                + recv_buf[sl].astype(jnp.float32)
            )
        for i in range(NCHUNK):
            rdma(i).wait_send()

    return pl.pallas_call(
        body,
        out_shape=jax.ShapeDtypeStruct((m, N_OUT), jnp.float32),
        in_specs=[pl.BlockSpec(memory_space=pltpu.VMEM)],
        out_specs=pl.BlockSpec(memory_space=pltpu.VMEM),
        scratch_shapes=[
            pltpu.VMEM((m, N_OUT), jnp.bfloat16),
            pltpu.SemaphoreType.DMA((NCHUNK,)),
            pltpu.SemaphoreType.DMA((NCHUNK,)),
        ],
        compiler_params=pltpu.CompilerParams(collective_id=0),
    )(xb)
